# Initial kernel scaffold; baseline (speedup 1.0000x reference)
import jax
import jax.numpy as jnp
from jax.experimental import pallas as pl


def kernel(x, weights, W1, b1, W2, b2, W3, b3, W4, b4, g1, be1, g2, be2, g3, be3, g4, be4, Wd, bd, edge_index, batch):
    raise NotImplementedError("write your pallas kernel here")


# speedup vs baseline: 1.3263x; 1.3263x over previous
import functools

import jax
import jax.numpy as jnp
from jax.experimental import pallas as pl
from jax.experimental.pallas import tpu as pltpu

_MAXN = 2848
_EPS = 1e-5
_B = 4


def _layer_body(h3_ref, w_ref, b_ref, g_ref, be_ref, o_ref):
    y = jnp.dot(h3_ref[...], w_ref[...],
                preferred_element_type=jnp.float32,
                precision=jax.lax.Precision.HIGHEST)
    y = y + b_ref[...]
    m = jnp.mean(y, axis=0, keepdims=True)
    v = jnp.mean((y - m) ** 2, axis=0, keepdims=True)
    yn = g_ref[...] * (y - m) * jax.lax.rsqrt(v + _EPS) + be_ref[...]
    o_ref[...] = jnp.maximum(yn, 0.0)


def _fused_layer(h3, wc, b, g, be):
    n = h3.shape[0]
    u = wc.shape[1]
    return pl.pallas_call(
        _layer_body,
        out_shape=jax.ShapeDtypeStruct((n, u), jnp.float32),
    )(h3, wc, b.reshape(1, -1), g.reshape(1, -1), be.reshape(1, -1))


def _head_body(h4_ref, z_ref, batch_ref, pos_ref, bd_ref, o_ref):
    keep = (pos_ref[...] < _MAXN).astype(jnp.float32)
    rows = jnp.sum(h4_ref[...] * z_ref[...], axis=1, keepdims=True) * keep
    batchv = batch_ref[...]
    cols = []
    for b in range(_B):
        sel = (batchv == b).astype(jnp.float32)
        cols.append(jnp.sum(rows * sel).reshape(1, 1))
    o_ref[...] = jnp.concatenate(cols, axis=1) + bd_ref[...]


def _head(h4, z, batch, pos, bd):
    n = h4.shape[0]
    out = pl.pallas_call(
        _head_body,
        out_shape=jax.ShapeDtypeStruct((1, _B), jnp.float32),
    )(h4, z, batch.reshape(n, 1), pos.reshape(n, 1), bd.reshape(1, 1))
    return out.reshape(_B)


def _spmm(z, src, dst, w, n):
    return jax.ops.segment_sum(w[:, None] * z[src], dst, num_segments=n)


def kernel(x, weights, W1, b1, W2, b2, W3, b3, W4, b4,
           g1, be1, g2, be2, g3, be3, g4, be4, Wd, bd, edge_index, batch):
    n = x.shape[0]
    src, dst = edge_index[0], edge_index[1]

    deg = jax.ops.segment_sum(weights, src, num_segments=n)
    dis = jnp.where(deg > 0, jax.lax.rsqrt(jnp.where(deg > 0, deg, 1.0)), 0.0)
    disc = dis[:, None]

    h = x
    for (W, b, g, be) in ((W1, b1, g1, be1), (W2, b2, g2, be2),
                          (W3, b3, g3, be3), (W4, b4, g4, be4)):
        s1 = _spmm(disc * h, src, dst, weights, n)
        t1 = -disc * s1
        s2raw = _spmm(disc * t1, src, dst, weights, n)
        s2 = -disc * s2raw
        h3 = jnp.concatenate([h, t1, s2], axis=1)
        wc = jnp.concatenate([W[0] - W[2], W[1], 2.0 * W[2]], axis=0)
        h = _fused_layer(h3, wc, b, g, be)

    counts = jnp.bincount(batch, length=_B)
    starts = jnp.concatenate(
        [jnp.zeros((1,), counts.dtype), jnp.cumsum(counts)[:-1]])
    pos = jnp.arange(n, dtype=jnp.int32) - starts[batch].astype(jnp.int32)
    wdr = Wd.reshape(_MAXN, -1)
    z = wdr[jnp.clip(pos, 0, _MAXN - 1)]
    return _head(h, z, batch, pos, bd)


# speedup vs baseline: 2.7188x; 2.0498x over previous
import functools

import jax
import jax.numpy as jnp
from jax import lax
from jax.experimental import pallas as pl
from jax.experimental.pallas import tpu as pltpu
from jax.experimental.pallas import tpu_sc as plsc

_NC, _NS, _LANES = 2, 16, 16
_NW = _NC * _NS
_G = 128
_D = 128
_MAXN = 2848
_EPS = 1e-5
_B = 4


def _mesh():
    return plsc.VectorSubcoreMesh(core_axis_name="c", subcore_axis_name="s")


def _worker_id():
    return lax.axis_index("c") * _NS + lax.axis_index("s")


def _deg_body(gpw, rpt, srcg, wg, out, src_v, w_v, acc, zbuf):
    cid = lax.axis_index("c")
    sid = lax.axis_index("s")
    wid = cid * _NS + sid
    gb = wid * gpw
    z16 = jnp.zeros((_LANES,), jnp.float32)
    for o in range(0, rpt, _LANES):
        zbuf[pl.ds(o, _LANES)] = z16
    pltpu.sync_copy(zbuf, acc.at[pl.ds(sid * rpt, rpt)])
    plsc.subcore_barrier()

    def grp(g, carry):
        pltpu.sync_copy(srcg.at[gb + g], src_v)
        pltpu.sync_copy(wg.at[gb + g], w_v)
        pltpu.sync_copy(w_v, acc.at[src_v], add=True)
        return carry

    lax.fori_loop(0, gpw, grp, 0)
    plsc.subcore_barrier()
    pltpu.sync_copy(acc.at[pl.ds(sid * rpt, rpt)],
                    out.at[cid].at[pl.ds(sid * rpt, rpt)])


def _deg_kernel(srcg, wg, npad):
    gpw = srcg.shape[0] // _NW
    rpt = npad // _NS
    k = functools.partial(
        pl.kernel,
        out_type=jax.ShapeDtypeStruct((_NC, npad), jnp.float32),
        mesh=_mesh(),
        scratch_types=[
            pltpu.VMEM((_G,), jnp.int32),
            pltpu.VMEM((_G,), jnp.float32),
            pltpu.VMEM_SHARED((npad,), jnp.float32),
            pltpu.VMEM((rpt,), jnp.float32),
        ],
    )(functools.partial(_deg_body, gpw, rpt))
    return k(srcg, wg)


def _norm_body(gpw, disr, srcg, dstg, wg, out, dis_v, src_v, dst_v, w_v, nv_v):
    wid = _worker_id()
    gb = wid * gpw
    pltpu.sync_copy(disr, dis_v)
    pltpu.sync_copy(srcg.at[pl.ds(gb, gpw)], src_v)
    pltpu.sync_copy(dstg.at[pl.ds(gb, gpw)], dst_v)
    pltpu.sync_copy(wg.at[pl.ds(gb, gpw)], w_v)

    def gat(i16):
        hi = lax.shift_right_logical(i16, 7)
        lo = lax.bitwise_and(i16, 127)
        return plsc.load_gather(dis_v, [hi, lo])

    def grp(g, carry):
        for o in range(0, _G, _LANES):
            s16 = src_v[g, pl.ds(o, _LANES)]
            d16 = dst_v[g, pl.ds(o, _LANES)]
            w16 = w_v[g, pl.ds(o, _LANES)]
            nv_v[g, pl.ds(o, _LANES)] = -(gat(s16) * w16 * gat(d16))
        return carry

    lax.fori_loop(0, gpw, grp, 0)
    pltpu.sync_copy(nv_v, out.at[pl.ds(gb, gpw)])


def _norm_kernel(dis, srcg, dstg, wg):
    gpw = srcg.shape[0] // _NW
    npad = dis.shape[0]
    k = functools.partial(
        pl.kernel,
        out_type=jax.ShapeDtypeStruct((srcg.shape[0], _G), jnp.float32),
        mesh=_mesh(),
        compiler_params=pltpu.CompilerParams(needs_layout_passes=False),
        scratch_types=[
            pltpu.VMEM((npad // _G, _G), jnp.float32),
            pltpu.VMEM((gpw, _G), jnp.int32),
            pltpu.VMEM((gpw, _G), jnp.int32),
            pltpu.VMEM((gpw, _G), jnp.float32),
            pltpu.VMEM((gpw, _G), jnp.float32),
        ],
    )(functools.partial(_norm_body, gpw))
    return k(dis.reshape(npad // _G, _G), srcg, dstg, wg)


_U = 64


def _prop_body(gpw, rpt, z, srcg, dstg, nvg, out,
               src_v, dst_v, nv_v, rows, acc, sem):
    cid = lax.axis_index("c")
    sid = lax.axis_index("s")
    wid = cid * _NS + sid
    gb = wid * gpw
    z16 = jnp.zeros((_LANES,), jnp.float32)

    def zr(e, carry):
        for c in range(_D // _LANES):
            rows[e, pl.ds(c * _LANES, _LANES)] = z16
        return carry

    lax.fori_loop(0, _G, zr, 0)
    for off in range(0, rpt, _G):
        pltpu.sync_copy(rows, acc.at[pl.ds(sid * rpt + off, _G)])
    plsc.subcore_barrier()

    def grp(g, carry):
        pltpu.sync_copy(srcg.at[gb + g], src_v)
        pltpu.sync_copy(dstg.at[gb + g], dst_v)
        pltpu.sync_copy(nvg.at[gb + g], nv_v)
        pltpu.async_copy(z.at[src_v], rows, sem).wait()
        for ci in range(_G // _LANES):
            nv16 = nv_v[pl.ds(ci * _LANES, _LANES)]
            e0 = ci * _LANES
            for l in range(_LANES):
                wv = nv16[l]
                for c in range(_D // _LANES):
                    sl = pl.ds(c * _LANES, _LANES)
                    rows[e0 + l, sl] = rows[e0 + l, sl] * wv
        pltpu.sync_copy(rows, acc.at[dst_v], add=True)
        return carry

    lax.fori_loop(0, gpw, grp, 0)
    plsc.subcore_barrier()
    pltpu.sync_copy(acc.at[pl.ds(sid * rpt, rpt)],
                    out.at[cid].at[pl.ds(sid * rpt, rpt)])


def _prop_kernel(z, srcg, dstg, nvg, npad):
    assert z.shape[1] == _D
    gpw = srcg.shape[0] // _NW
    rpt = npad // _NS
    k = functools.partial(
        pl.kernel,
        out_type=jax.ShapeDtypeStruct((_NC, npad, _D), jnp.float32),
        mesh=_mesh(),
        compiler_params=pltpu.CompilerParams(needs_layout_passes=False),
        scratch_types=[
            pltpu.VMEM((_G,), jnp.int32),
            pltpu.VMEM((_G,), jnp.int32),
            pltpu.VMEM((_G,), jnp.float32),
            pltpu.VMEM((_G, _D), jnp.float32),
            pltpu.VMEM_SHARED((npad, _D), jnp.float32),
            pltpu.SemaphoreType.DMA,
        ],
    )(functools.partial(_prop_body, gpw, rpt))
    return k(z, srcg, dstg, nvg)


def _zg_body(ng, wdr, idxg, out, idx_v, rows, sem):
    wid = _worker_id()
    extra = ng - 2 * _NW
    b0 = jnp.where(wid < extra, 3 * wid, 2 * wid + extra)

    def dog(g):
        pltpu.sync_copy(idxg.at[g], idx_v)
        pltpu.async_copy(wdr.at[idx_v], rows, sem).wait()
        pltpu.sync_copy(rows, out.at[pl.ds(g * _G, _G)])

    dog(b0)
    dog(b0 + 1)

    @pl.when(wid < extra)
    def _():
        dog(b0 + 2)


def _zg_kernel(wdr, idxg):
    ng = idxg.shape[0]
    k = functools.partial(
        pl.kernel,
        out_type=jax.ShapeDtypeStruct((ng * _G, _D), jnp.float32),
        mesh=_mesh(),
        scratch_types=[
            pltpu.VMEM((_G,), jnp.int32),
            pltpu.VMEM((_G, _D), jnp.float32),
            pltpu.SemaphoreType.DMA,
        ],
    )(functools.partial(_zg_body, ng))
    return k(wdr, idxg)


def _dis_body(degp_ref, o_ref):
    deg = degp_ref[0:1, :] + degp_ref[1:2, :]
    safe = jnp.where(deg > 0, deg, 1.0)
    o_ref[...] = jnp.where(deg > 0, jax.lax.rsqrt(safe), 0.0)


def _dis_kernel(degp):
    npad = degp.shape[1]
    out = pl.pallas_call(
        _dis_body,
        out_shape=jax.ShapeDtypeStruct((1, npad), jnp.float32),
    )(degp)
    return out.reshape(npad)


def _mid_body(n, p_ref, o_ref):
    o_ref[...] = p_ref[0, :n, :] + p_ref[1, :n, :]


def _mid_kernel(part, n):
    return pl.pallas_call(
        functools.partial(_mid_body, n),
        out_shape=jax.ShapeDtypeStruct((n, _D), jnp.float32),
    )(part)


_BS = 1000


def _ymom_tail(i, nsteps, n, y, y_ref, mom_ref, acc_ref):
    y_ref[...] = y
    part = jnp.concatenate([jnp.sum(y, axis=0, keepdims=True),
                            jnp.sum(y * y, axis=0, keepdims=True)], axis=0)

    @pl.when(i == 0)
    def _():
        acc_ref[...] = jnp.zeros_like(acc_ref)

    acc_ref[...] += part

    @pl.when(i == nsteps - 1)
    def _():
        mom_ref[...] = acc_ref[...] * (1.0 / n)


def _ymom_body(nsteps, n, h_ref, t1_ref, s2_ref, w0_ref, w1_ref, w2_ref,
               b_ref, y_ref, mom_ref, acc_ref):
    hp = jax.lax.Precision.HIGHEST
    y = jnp.dot(h_ref[...], w0_ref[...],
                preferred_element_type=jnp.float32, precision=hp)
    y = y + jnp.dot(t1_ref[...], w1_ref[...],
                    preferred_element_type=jnp.float32, precision=hp)
    y = y + jnp.dot(s2_ref[...], w2_ref[...],
                    preferred_element_type=jnp.float32, precision=hp)
    y = y + b_ref[...]
    _ymom_tail(pl.program_id(0), nsteps, n, y, y_ref, mom_ref, acc_ref)


def _row_spec(d):
    return pl.BlockSpec((_BS, d), lambda i: (i, 0))


def _full_spec(shape):
    return pl.BlockSpec(shape, lambda i: tuple(0 for _ in shape))


def _ymom_outs(n, u, nsteps):
    return dict(
        grid=(nsteps,),
        out_specs=[_row_spec(u), _full_spec((2, u))],
        out_shape=[jax.ShapeDtypeStruct((n, u), jnp.float32),
                   jax.ShapeDtypeStruct((2, u), jnp.float32)],
        scratch_shapes=[pltpu.VMEM((2, u), jnp.float32)],
    )


def _bn_body(y_ref, mom_ref, g_ref, be_ref, o_ref):
    m = mom_ref[0:1, :]
    v = mom_ref[1:2, :] - m * m
    yn = g_ref[...] * (y_ref[...] - m) * jax.lax.rsqrt(v + _EPS) + be_ref[...]
    o_ref[...] = jnp.maximum(yn, 0.0)


def _bn_kernel(y, mom, g, be):
    n, u = y.shape
    nsteps = n // _BS
    return pl.pallas_call(
        _bn_body,
        grid=(nsteps,),
        in_specs=[_row_spec(u), _full_spec((2, u)),
                  _full_spec((1, u)), _full_spec((1, u))],
        out_specs=_row_spec(u),
        out_shape=jax.ShapeDtypeStruct((n, u), jnp.float32),
    )(y, mom, g.reshape(1, -1), be.reshape(1, -1))


def _layer_kernel(h, t1, s2, w0, w1, w2, b, g, be):
    n = h.shape[0]
    u = w0.shape[1]
    nsteps = n // _BS
    y, mom = pl.pallas_call(
        functools.partial(_ymom_body, nsteps, n),
        in_specs=[_row_spec(_D), _row_spec(_D), _row_spec(_D),
                  _full_spec((_D, u)), _full_spec((_D, u)),
                  _full_spec((_D, u)), _full_spec((1, u))],
        **_ymom_outs(n, u, nsteps),
    )(h, t1, s2, w0, w1, w2, b.reshape(1, -1))
    return _bn_kernel(y, mom, g, be)


def _pos_body(n, batch_ref, pos_ref, posc_ref):
    bv = batch_ref[...]
    iota = jax.lax.broadcasted_iota(jnp.int32, (n, 1), 0)
    start = jnp.zeros((n, 1), jnp.int32)
    acc = jnp.zeros((), jnp.int32)
    for b in range(1, _B):
        acc = acc + jnp.sum((bv == b - 1).astype(jnp.int32))
        start = start + acc * (bv == b).astype(jnp.int32)
    pos = iota - start
    pos_ref[...] = pos
    posc_ref[...] = jnp.clip(pos, 0, _MAXN - 1)


def _pos_kernel(batch):
    n = batch.shape[0]
    return pl.pallas_call(
        functools.partial(_pos_body, n),
        out_shape=[jax.ShapeDtypeStruct((n, 1), jnp.int32),
                   jax.ShapeDtypeStruct((n, 1), jnp.int32)],
    )(batch.reshape(n, 1))


def _head_body(h4_ref, z_ref, batch_ref, pos_ref, bd_ref, o_ref):
    keep = (pos_ref[...] < _MAXN).astype(jnp.float32)
    rows = jnp.sum(h4_ref[...] * z_ref[...], axis=1, keepdims=True) * keep
    batchv = batch_ref[...]
    cols = []
    for b in range(_B):
        sel = (batchv == b).astype(jnp.float32)
        cols.append(jnp.sum(rows * sel).reshape(1, 1))
    o_ref[...] = jnp.concatenate(cols, axis=1) + bd_ref[...]


def _head(h4, z, batch, pos, bd):
    n = h4.shape[0]
    out = pl.pallas_call(
        _head_body,
        out_shape=jax.ShapeDtypeStruct((1, _B), jnp.float32),
    )(h4, z, batch.reshape(n, 1), pos, bd.reshape(1, 1))
    return out.reshape(_B)


def _pad_to(x, m, value=0):
    pad = (-x.shape[0]) % m
    if pad == 0:
        return x
    return jnp.pad(x, ((0, pad),), constant_values=value)


def kernel(x, weights, W1, b1, W2, b2, W3, b3, W4, b4,
           g1, be1, g2, be2, g3, be3, g4, be4, Wd, bd, edge_index, batch):
    n = x.shape[0]
    u = W1.shape[2]
    src, dst = edge_index[0], edge_index[1]

    epm = _NW * _G * 8
    srcg = _pad_to(src, epm).reshape(-1, _G)
    dstg = _pad_to(dst, epm).reshape(-1, _G)
    wg = _pad_to(weights, epm).reshape(-1, _G)
    npad = ((n + 255) // 256) * 256

    degp = _deg_kernel(srcg, wg, npad)
    dis = _dis_kernel(degp)
    nvg = _norm_kernel(dis, srcg, dstg, wg)

    def _prop(z):
        return _mid_kernel(_prop_kernel(z, srcg, dstg, nvg, npad), n)

    def _tab(v64):
        return jnp.pad(v64, ((0, 0), (0, _D - _U)))

    def _wpad(w):
        return w if w.shape[0] == _D else jnp.pad(
            w, ((0, _D - w.shape[0]), (0, 0)))

    htab = x
    h = None
    for (W, b, g, be) in ((W1, b1, g1, be1), (W2, b2, g2, be2),
                          (W3, b3, g3, be3), (W4, b4, g4, be4)):
        t1 = _prop(htab)
        s2 = _prop(t1)
        h = _layer_kernel(htab, t1, s2, _wpad(W[0] - W[2]), _wpad(W[1]),
                          _wpad(2.0 * W[2]), b, g, be)
        htab = _tab(h)

    pos, posc = _pos_kernel(batch)
    idxg = _pad_to(posc.reshape(n), _G).reshape(-1, _G)
    wdr = jnp.pad(Wd.reshape(_MAXN, u), ((0, 0), (0, _D - u)))
    z = _zg_kernel(wdr, idxg)[:n]
    return _head(htab, z, batch, pos, bd)
